# baseline (speedup 1.0000x reference)
import functools

import jax
import jax.numpy as jnp
from jax import lax
from jax.experimental import pallas as pl
from jax.experimental.pallas import tpu as pltpu
from jax.experimental.pallas import tpu_sc as plsc


@functools.lru_cache(maxsize=None)
def _build(B, D, N, NC, NS):
    NW = NC * NS
    b_per_w = B // NW
    C = 128
    NCH = b_per_w // C
    NSLOT = 3

    mesh = plsc.VectorSubcoreMesh(core_axis_name="c", subcore_axis_name="s")

    @functools.partial(
        pl.kernel,
        mesh=mesh,
        out_type=(
            jax.ShapeDtypeStruct((B, D), jnp.float32),
            jax.ShapeDtypeStruct((B, D), jnp.float32),
            jax.ShapeDtypeStruct((B,), jnp.float32),
        ),
        scratch_types=(
            [pltpu.VMEM((NCH, C), jnp.int32),
             pltpu.VMEM((b_per_w,), jnp.float32),
             pltpu.VMEM((b_per_w,), jnp.float32)]
            + [pltpu.VMEM((C, D), jnp.float32) for _ in range(2 * NSLOT)]
            + [pltpu.SemaphoreType.DMA for _ in range(4 * NSLOT)]
            + [pltpu.SemaphoreType.DMA]
        ),
    )
    def k(nids_hbm, ts_hbm, left_hbm, right_hbm, pts_hbm,
          hl_out, hr_out, pts_out,
          idx_v, ts_v, pts_v, *rest):
        rowbufs = rest[:2 * NSLOT]
        glsems = rest[2 * NSLOT:3 * NSLOT]
        grsems = rest[3 * NSLOT:4 * NSLOT]
        wlsems = rest[4 * NSLOT:5 * NSLOT]
        wrsems = rest[5 * NSLOT:6 * NSLOT]
        psem = rest[6 * NSLOT]
        bufs = tuple((rowbufs[2 * s], rowbufs[2 * s + 1])
                     for s in range(NSLOT))
        wid = lax.axis_index("s") * NC + lax.axis_index("c")
        base = wid * b_per_w
        pltpu.sync_copy(nids_hbm.at[wid], idx_v)

        def fire_gather(j, slot):
            ij = idx_v.at[j]
            l, r = bufs[slot]
            return (pltpu.async_copy(left_hbm.at[ij], l, glsems[slot]),
                    pltpu.async_copy(right_hbm.at[ij], r, grsems[slot]))

        AHEAD = NSLOT - 1
        pending_g = [None] * NSLOT
        pending_w = [None] * NSLOT
        for j0 in range(min(AHEAD, NCH)):
            pending_g[j0 % NSLOT] = fire_gather(j0, j0 % NSLOT)

        pcps = [pltpu.async_copy(pts_hbm.at[idx_v.at[j]],
                                 pts_v.at[pl.ds(j * C, C)], psem)
                for j in range(NCH)]
        tscp = pltpu.async_copy(ts_hbm.at[pl.ds(base, b_per_w)], ts_v, psem)
        for cp in pcps:
            cp.wait()
        tscp.wait()
        for i in range(b_per_w // 16):
            sl = pl.ds(i * 16, 16)
            pts_v[sl] = jnp.minimum(pts_v[sl], ts_v[sl])
        pwcp = pltpu.async_copy(pts_v, pts_out.at[pl.ds(base, b_per_w)], psem)

        for j in range(NCH):
            slot = j % NSLOT
            f = j + AHEAD
            if f < NCH:
                fslot = f % NSLOT
                if pending_w[fslot] is not None:
                    for cp in pending_w[fslot]:
                        cp.wait()
                    pending_w[fslot] = None
                pending_g[fslot] = fire_gather(f, fslot)
            gl, gr = pending_g[slot]
            l, r = bufs[slot]
            o = pl.ds(base + j * C, C)
            gl.wait()
            wl = pltpu.async_copy(l, hl_out.at[o], wlsems[slot])
            gr.wait()
            wr = pltpu.async_copy(r, hr_out.at[o], wrsems[slot])
            pending_w[slot] = (wl, wr)
        pwcp.wait()
        for pw in pending_w:
            if pw is not None:
                for cp in pw:
                    cp.wait()

    return k, NW, NCH, C


def kernel(nids, ts, left_weight, right_weight, prev_ts_table):
    B, = nids.shape
    N, D = left_weight.shape
    info = plsc.get_sparse_core_info()
    k, NW, NCH, C = _build(B, D, N, info.num_cores, info.num_subcores)
    nids3 = nids.astype(jnp.int32).reshape(NW, NCH, C)
    h_left, h_right, prev_ts = k(nids3, ts, left_weight, right_weight,
                                 prev_ts_table)
    return (h_left, h_right, prev_ts)


# speedup vs baseline: 1.0208x; 1.0208x over previous
import functools

import jax
import jax.numpy as jnp
from jax import lax
from jax.experimental import pallas as pl
from jax.experimental.pallas import tpu as pltpu
from jax.experimental.pallas import tpu_sc as plsc


@functools.lru_cache(maxsize=None)
def _build(B, D, N, NC, NS):
    NW = NC * NS
    b_per_w = B // NW
    C = 128
    NCH = b_per_w // C
    NSLOT = 3

    mesh = plsc.VectorSubcoreMesh(core_axis_name="c", subcore_axis_name="s")

    @functools.partial(
        pl.kernel,
        mesh=mesh,
        out_type=(
            jax.ShapeDtypeStruct((B, D), jnp.float32),
            jax.ShapeDtypeStruct((B, D), jnp.float32),
            jax.ShapeDtypeStruct((B,), jnp.float32),
        ),
        scratch_types=(
            [pltpu.VMEM((NCH, C), jnp.int32),
             pltpu.VMEM((b_per_w,), jnp.float32),
             pltpu.VMEM((b_per_w,), jnp.float32)]
            + [pltpu.VMEM((C, D), jnp.float32) for _ in range(2 * NSLOT)]
            + [pltpu.SemaphoreType.DMA for _ in range(2 * NSLOT)]
            + [pltpu.SemaphoreType.DMA]
        ),
    )
    def k(nids_hbm, ts_hbm, left_hbm, right_hbm, pts_hbm,
          hl_out, hr_out, pts_out,
          idx_v, ts_v, pts_v, *rest):
        rowbufs = rest[:2 * NSLOT]
        gsems = rest[2 * NSLOT:3 * NSLOT]
        wsems = rest[3 * NSLOT:4 * NSLOT]
        psem = rest[4 * NSLOT]
        bufs = tuple((rowbufs[2 * s], rowbufs[2 * s + 1])
                     for s in range(NSLOT))
        wid = lax.axis_index("s") * NC + lax.axis_index("c")
        base = wid * b_per_w
        pltpu.sync_copy(nids_hbm.at[wid], idx_v)

        def fire_gather(j, slot):
            ij = idx_v.at[j]
            l, r = bufs[slot]
            return (pltpu.async_copy(left_hbm.at[ij], l, gsems[slot]),
                    pltpu.async_copy(right_hbm.at[ij], r, gsems[slot]))

        pcps = [pltpu.async_copy(pts_hbm.at[idx_v.at[j]],
                                 pts_v.at[pl.ds(j * C, C)], psem)
                for j in range(NCH)]
        tscp = pltpu.async_copy(ts_hbm.at[pl.ds(base, b_per_w)], ts_v, psem)

        AHEAD = NSLOT - 1
        pending_g = [None] * NSLOT
        pending_w = [None] * NSLOT
        for j0 in range(min(AHEAD, NCH)):
            pending_g[j0 % NSLOT] = fire_gather(j0, j0 % NSLOT)
        for cp in pcps:
            cp.wait()
        tscp.wait()
        for i in range(b_per_w // 16):
            sl = pl.ds(i * 16, 16)
            pts_v[sl] = jnp.minimum(pts_v[sl], ts_v[sl])
        pwcp = pltpu.async_copy(pts_v, pts_out.at[pl.ds(base, b_per_w)], psem)

        for j in range(NCH):
            slot = j % NSLOT
            f = j + AHEAD
            if f < NCH:
                fslot = f % NSLOT
                if pending_w[fslot] is not None:
                    for cp in pending_w[fslot]:
                        cp.wait()
                    pending_w[fslot] = None
                pending_g[fslot] = fire_gather(f, fslot)
            for cp in pending_g[slot]:
                cp.wait()
            l, r = bufs[slot]
            o = pl.ds(base + j * C, C)
            pending_w[slot] = (
                pltpu.async_copy(l, hl_out.at[o], wsems[slot]),
                pltpu.async_copy(r, hr_out.at[o], wsems[slot]))
        pwcp.wait()
        for pw in pending_w:
            if pw is not None:
                for cp in pw:
                    cp.wait()

    return k, NW, NCH, C


def kernel(nids, ts, left_weight, right_weight, prev_ts_table):
    B, = nids.shape
    N, D = left_weight.shape
    info = plsc.get_sparse_core_info()
    k, NW, NCH, C = _build(B, D, N, info.num_cores, info.num_subcores)
    nids3 = nids.astype(jnp.int32).reshape(NW, NCH, C)
    h_left, h_right, prev_ts = k(nids3, ts, left_weight, right_weight,
                                 prev_ts_table)
    return (h_left, h_right, prev_ts)
